# Initial kernel scaffold; baseline (speedup 1.0000x reference)
import jax
import jax.numpy as jnp
from jax.experimental import pallas as pl


def kernel(node_f, node_x, edge_index, edge_attr, graph_ids, params):
    raise NotImplementedError("write your pallas kernel here")


# speedup vs baseline: 13.9719x; 13.9719x over previous
import functools

import jax
import jax.numpy as jnp
import numpy as np
from jax import lax
from jax.experimental import pallas as pl
from jax.experimental.pallas import tpu as pltpu
from jax.experimental.pallas import tpu_sc as plsc

N = 10000
E = 320000
D = 128
H = 4
DH = 32
L = 4
ED = 5
VC = 8
G = 512

NP_ = 10240
EP = 327680
CE = EP // 128
BN = 512
BE = 512
NC, NS = 2, 16
NW = NC * NS
K_POOL = 128

F32 = jnp.float32


def _np_consts():
    c = {}
    Pr16 = np.zeros((16, 32), np.float32)
    PrE = np.zeros((16, 128), np.float32)
    for o in range(8):
        for dd in range(3):
            Pr16[dd, 3 * o + dd] = 1
            PrE[dd, 3 * o + dd] = 1
    c['Pr16'] = Pr16
    c['PrE'] = PrE
    M03 = np.zeros((1, 16), np.float32); M03[0, :3] = 1
    Mc3 = np.zeros((1, 16), np.float32); Mc3[0, 3] = 1
    Ones3 = np.zeros((16, 16), np.float32); Ones3[:3, :] = 1
    S8 = np.zeros((8, 16), np.float32)
    for i in range(5):
        S8[i, 4 + i] = 1
    c['M03'], c['Mc3'], c['Ones3'], c['S8'] = M03, Mc3, Ones3, S8
    Gh = np.zeros((128, 128), np.float32)
    Gb = np.zeros((128, 128), np.float32)
    for h in range(4):
        Gh[h * 32:(h + 1) * 32, h] = 1
        Gb[h, h * 32:(h + 1) * 32] = 1
    c['Gh'], c['Gb'] = Gh, Gb
    Pg = np.zeros((128, 128), np.float32)
    for o in range(8):
        for dd in range(3):
            Pg[o, 3 * o + dd] = 1
    c['Pg'] = Pg
    Db32 = np.zeros((32, 128), np.float32)
    for h in range(4):
        Db32[h, h * 32:(h + 1) * 32] = 1
    c['Db32'] = Db32
    Dv32 = np.zeros((32, 32), np.float32)
    for j in range(24):
        Dv32[4 + j, j] = 1
    c['Dv32'] = Dv32
    Gmat = np.zeros((32, 32), np.float32)
    Gf = np.zeros((32, 32), np.float32)
    for a in range(24):
        Gf[a, a // 3] = 1
        for b in range(24):
            if a // 3 == b // 3:
                Gmat[a, b] = 1
    c['Gmat'], c['Gf'] = Gmat, Gf
    return c


_C = _np_consts()


def _ln(x):
    m = jnp.mean(x, axis=-1, keepdims=True)
    v = jnp.mean((x - m) * (x - m), axis=-1, keepdims=True)
    return (x - m) / jnp.sqrt(v + 1e-5)


def _dot(a, b):
    return jnp.dot(a, b, preferred_element_type=F32)


def _full(spec_shape):
    return pl.BlockSpec(spec_shape, lambda i: tuple(0 for _ in spec_shape))


def _rows(bs, w):
    return pl.BlockSpec((bs, w), lambda i: (i, 0))


def _prelude_body(f8, xp, eW8, eb, qW1, qb1, qW2, qb2, Pr16, ve24,
                  sc_o, q_o, vec_o):
    sc = _dot(f8[...], eW8[...]) + eb[...]
    sc_o[...] = sc
    q_o[...] = _dot(jax.nn.relu(_ln(_dot(sc, qW1[...]) + qb1[...])),
                    qW2[...]) + qb2[...]
    vec_o[...] = _dot(xp[:, :16], Pr16[...]) * ve24[...]


def _prelude_call(f8, xp, eW8, eb, qW1, qb1, qW2, qb2, ve24):
    n = NP_ // BN
    return pl.pallas_call(
        _prelude_body,
        grid=(n,),
        in_specs=[_rows(BN, 8), _rows(BN, 128), _full((8, 128)), _full((1, 128)),
                  _full((128, 128)), _full((1, 128)), _full((128, 128)),
                  _full((1, 128)), _full((16, 32)), _full((1, 32))],
        out_specs=[_rows(BN, 128), _rows(BN, 128), _rows(BN, 32)],
        out_shape=[jax.ShapeDtypeStruct((NP_, 128), F32),
                   jax.ShapeDtypeStruct((NP_, 128), F32),
                   jax.ShapeDtypeStruct((NP_, 32), F32)],
    )(f8, xp, eW8, eb, qW1, qb1, qW2, qb2, jnp.asarray(_C['Pr16']), ve24)


def _geo_body(xs, xd, ea8, M03, Mc3, Ones3, S8, out):
    rel = xs[:, :16] - xd[:, :16]
    d2b = _dot(rel * rel, Ones3[...])
    dist = jnp.sqrt(d2b + 1e-12)
    ru = rel / (dist + 1e-8)
    out[...] = ru * M03[...] + dist * Mc3[...] + _dot(ea8[...], S8[...])


def _geo_call(xs, xd, ea8):
    n = EP // BE
    return pl.pallas_call(
        _geo_body,
        grid=(n,),
        in_specs=[_rows(BE, 128), _rows(BE, 128), _rows(BE, 8),
                  _full((1, 16)), _full((1, 16)), _full((16, 16)),
                  _full((8, 16))],
        out_specs=[_rows(BE, 16)],
        out_shape=[jax.ShapeDtypeStruct((EP, 16), F32)],
    )(xs, xd, ea8, jnp.asarray(_C['M03']), jnp.asarray(_C['Mc3']),
      jnp.asarray(_C['Ones3']), jnp.asarray(_C['S8']))[0]


def _edge_body(sg, qg, egeo, W1s, W1e, b1, kW2, kb2, vW2, vb2, gWp, gbp,
               Gh, Gb, Pg, PrE, m_o, r_o):
    kv = _dot(sg[...], W1s[...]) + _dot(egeo[...], W1e[...]) + b1[...]
    hk = jax.nn.relu(_ln(kv[:, :128]))
    hv = jax.nn.relu(_ln(kv[:, 128:]))
    k = _dot(hk, kW2[...]) + kb2[...]
    v = _dot(hv, vW2[...]) + vb2[...]
    ex128 = jnp.exp(_dot(qg[...] * k, Gh[...]) * (1.0 / np.sqrt(DH)))
    m_o[...] = _dot(ex128, Gb[...]) * v
    gate = jnp.tanh(_dot(v, gWp[...]) + gbp[...])
    vm = _dot(gate, Pg[...]) * _dot(egeo[...], PrE[...])
    r_o[...] = jnp.concatenate(
        [ex128[:, :4], vm[:, :24], jnp.zeros((BE, 100), F32)], axis=1)


def _edge_call(sg, qg, egeo, W1s, W1e, b1, kW2, kb2, vW2, vb2, gWp, gbp):
    n = EP // BE
    return pl.pallas_call(
        _edge_body,
        grid=(n,),
        in_specs=[_rows(BE, 128), _rows(BE, 128), _rows(BE, 16),
                  _full((128, 256)), _full((16, 256)), _full((1, 256)),
                  _full((128, 128)), _full((1, 128)), _full((128, 128)),
                  _full((1, 128)), _full((128, 128)), _full((1, 128)),
                  _full((128, 128)), _full((128, 128)), _full((128, 128)),
                  _full((16, 128))],
        out_specs=[_rows(BE, 128), _rows(BE, 128)],
        out_shape=[jax.ShapeDtypeStruct((EP, 128), F32),
                   jax.ShapeDtypeStruct((EP, 128), F32)],
    )(sg, qg, egeo, W1s, W1e, b1, kW2, kb2, vW2, vb2, gWp, gbp,
      jnp.asarray(_C['Gh']), jnp.asarray(_C['Gb']), jnp.asarray(_C['Pg']),
      jnp.asarray(_C['PrE']))


def _node_body(a0m, a1m, a0r, a1r, sc, vec, oW1, ob1, oW2, ob2,
               qW1, qb1, qW2, qb2, Db32, Dv32, Gmat, vsc24,
               sc_o, q_o, vec_o):
    msum = a0m[...] + a1m[...]
    s32 = a0r[...] + a1r[...]
    den = _dot(s32, Db32[...])
    agg = msum / (den + 1e-9)
    o1 = jax.nn.relu(_ln(_dot(agg, oW1[...]) + ob1[...]))
    sc2 = _ln(sc[...] + _dot(o1, oW2[...]) + ob2[...])
    sc_o[...] = sc2
    q_o[...] = _dot(jax.nn.relu(_ln(_dot(sc2, qW1[...]) + qb1[...])),
                    qW2[...]) + qb2[...]
    vec2 = vec[...] + _dot(s32, Dv32[...])
    gs = _dot(vec2 * vec2, Gmat[...])
    vec_o[...] = vec2 * vsc24[...] / (1.0 + jnp.sqrt(gs + 1e-12))


def _node_call(a0m, a1m, a0r, a1r, sc, vec, oW1, ob1, oW2, ob2,
               qW1, qb1, qW2, qb2, vsc24):
    n = NP_ // BN
    return pl.pallas_call(
        _node_body,
        grid=(n,),
        in_specs=[_rows(BN, 128), _rows(BN, 128), _rows(BN, 32),
                  _rows(BN, 32), _rows(BN, 128), _rows(BN, 32),
                  _full((128, 128)), _full((1, 128)), _full((128, 128)),
                  _full((1, 128)), _full((128, 128)), _full((1, 128)),
                  _full((128, 128)), _full((1, 128)),
                  _full((32, 128)), _full((32, 32)), _full((32, 32)),
                  _full((1, 32))],
        out_specs=[_rows(BN, 128), _rows(BN, 128), _rows(BN, 32)],
        out_shape=[jax.ShapeDtypeStruct((NP_, 128), F32),
                   jax.ShapeDtypeStruct((NP_, 128), F32),
                   jax.ShapeDtypeStruct((NP_, 32), F32)],
    )(a0m, a1m, a0r, a1r, sc, vec, oW1, ob1, oW2, ob2, qW1, qb1, qW2, qb2,
      jnp.asarray(_C['Db32']), jnp.asarray(_C['Dv32']),
      jnp.asarray(_C['Gmat']), vsc24)


def _featnode_body(sc, vec, W1s, W1v, b1, W2, b2, Gf, out):
    gs8 = _dot(vec[...] * vec[...], Gf[...])
    vinv = jnp.sqrt(gs8 + 1e-12)
    h1 = _dot(sc[...], W1s[...]) + _dot(vinv, W1v[...]) + b1[...]
    out[...] = _dot(jax.nn.relu(_ln(h1)), W2[...]) + b2[...] + sc[...]


def _featnode_call(sc, vec, W1s, W1v, b1, W2, b2):
    n = NP_ // BN
    return pl.pallas_call(
        _featnode_body,
        grid=(n,),
        in_specs=[_rows(BN, 128), _rows(BN, 32), _full((128, 128)),
                  _full((32, 128)), _full((1, 128)), _full((128, 128)),
                  _full((1, 128)), _full((32, 32))],
        out_specs=[_rows(BN, 128)],
        out_shape=[jax.ShapeDtypeStruct((NP_, 128), F32)],
    )(sc, vec, W1s, W1v, b1, W2, b2, jnp.asarray(_C['Gf']))[0]


def _pool_body(prows, out):
    for g in range(8):
        seg = prows[g * K_POOL:(g + 1) * K_POOL, :]
        mx = jnp.max(seg, axis=0, keepdims=True)
        out[g:g + 1, :] = jnp.where(mx < -1e29, 0.0, mx)


def _pool_call(prows):
    n = G // 8
    return pl.pallas_call(
        _pool_body,
        grid=(n,),
        in_specs=[pl.BlockSpec((8 * K_POOL, 128), lambda i: (i, 0))],
        out_specs=[_rows(8, 128)],
        out_shape=[jax.ShapeDtypeStruct((G, 128), F32)],
    )(prows)[0]


def _gmlp_body(pooled, W1, b1, W2p, b2p, out):
    h = jax.nn.relu(_ln(_dot(pooled[...], W1[...]) + b1[...]))
    out[...] = _dot(h, W2p[...]) + b2p[...]


def _gmlp_call(pooled, W1, b1, W2p, b2p):
    return pl.pallas_call(
        _gmlp_body,
        grid=(1,),
        in_specs=[_rows(G, 128), _full((128, 128)), _full((1, 128)),
                  _full((128, 128)), _full((1, 128))],
        out_specs=[_rows(G, 128)],
        out_shape=[jax.ShapeDtypeStruct((G, 128), F32)],
    )(pooled, W1, b1, W2p, b2p)[0]


def _mesh():
    return plsc.VectorSubcoreMesh(core_axis_name="c", subcore_axis_name="s",
                                  num_cores=NC, num_subcores=NS)


@functools.cache
def _make_gather2(n_idx_rows, wa, wb, rows_a, rows_b):
    per_w = n_idx_rows // NW

    @functools.partial(
        pl.kernel,
        out_type=(jax.ShapeDtypeStruct((n_idx_rows * 128, wa), F32),
                  jax.ShapeDtypeStruct((n_idx_rows * 128, wb), F32)),
        mesh=_mesh(),
        scratch_types=[pltpu.VMEM((per_w, 128), jnp.int32),
                       pltpu.VMEM((per_w, 128), jnp.int32),
                       pltpu.VMEM((128, wa), F32),
                       pltpu.VMEM((128, wb), F32),
                       pltpu.SemaphoreType.DMA,
                       pltpu.SemaphoreType.DMA],
    )
    def gather2(ta, ia, tb, ib, oa, ob, iva, ivb, ra, rb, sa, sb):
        wid = lax.axis_index("c") * NS + lax.axis_index("s")
        base = wid * per_w
        pltpu.sync_copy(ia.at[pl.ds(base, per_w)], iva)
        pltpu.sync_copy(ib.at[pl.ds(base, per_w)], ivb)

        def body(j, carry):
            ca = pltpu.async_copy(ta.at[iva.at[j]], ra, sa)
            cb = pltpu.async_copy(tb.at[ivb.at[j]], rb, sb)
            ca.wait()
            cb.wait()
            pltpu.sync_copy(ra, oa.at[pl.ds((base + j) * 128, 128)])
            pltpu.sync_copy(rb, ob.at[pl.ds((base + j) * 128, 128)])
            return carry

        lax.fori_loop(0, per_w, body, 0)

    return gather2


@functools.cache
def _make_gather1(n_idx_rows, w, rows_t):
    per_w = n_idx_rows // NW

    @functools.partial(
        pl.kernel,
        out_type=jax.ShapeDtypeStruct((n_idx_rows * 128, w), F32),
        mesh=_mesh(),
        scratch_types=[pltpu.VMEM((per_w, 128), jnp.int32),
                       pltpu.VMEM((128, w), F32),
                       pltpu.SemaphoreType.DMA],
    )
    def gather1(tab, idx, out, iv, rv, sem):
        wid = lax.axis_index("c") * NS + lax.axis_index("s")
        base = wid * per_w
        pltpu.sync_copy(idx.at[pl.ds(base, per_w)], iv)

        def body(j, carry):
            pltpu.async_copy(tab.at[iv.at[j]], rv, sem).wait()
            pltpu.sync_copy(rv, out.at[pl.ds((base + j) * 128, 128)])
            return carry

        lax.fori_loop(0, per_w, body, 0)

    return gather1


@functools.cache
def _make_scatter(w):
    per_w = CE // NW
    stripe = NP_ // NS

    @functools.partial(
        pl.kernel,
        out_type=jax.ShapeDtypeStruct((NC, NP_, w), F32),
        mesh=_mesh(),
        scratch_types=[pltpu.VMEM((per_w, 128), jnp.int32),
                       pltpu.VMEM((128, w), F32),
                       pltpu.VMEM_SHARED((NP_, w), F32)],
    )
    def scatter(mrows, didx, zm, om, iv, mv, accm):
        cid = lax.axis_index("c")
        sid = lax.axis_index("s")
        wid = cid * NS + sid
        base = wid * per_w
        pltpu.sync_copy(zm, accm.at[pl.ds(sid * stripe, stripe)])
        plsc.subcore_barrier()
        pltpu.sync_copy(didx.at[pl.ds(base, per_w)], iv)

        def body(j, carry):
            pltpu.sync_copy(mrows.at[pl.ds((base + j) * 128, 128)], mv)
            pltpu.sync_copy(mv, accm.at[iv.at[j]], add=True)
            return carry

        lax.fori_loop(0, per_w, body, 0)
        plsc.subcore_barrier()
        pltpu.sync_copy(accm.at[pl.ds(sid * stripe, stripe)],
                        om.at[cid, pl.ds(sid * stripe, stripe)])

    return scatter


def _rep3(v8):
    return jnp.pad(jnp.repeat(v8, 3), (0, 8)).reshape(1, 32)


def kernel(node_f, node_x, edge_index, edge_attr, graph_ids, params):
    p = params
    f6 = node_f[..., 0]
    f6 = f6.at[:, 5].set(f6[:, 5] / 9.0)
    f8 = jnp.pad(f6, ((0, NP_ - N), (0, 2)))
    xp = jnp.pad(node_x, ((0, NP_ - N), (0, 125)))
    src2 = jnp.pad(edge_index[0], (0, EP - E)).reshape(CE, 128).astype(jnp.int32)
    dst_pad = jnp.pad(edge_index[1], (0, EP - E), constant_values=N)
    dst2 = dst_pad.reshape(CE, 128).astype(jnp.int32)
    ea8 = jnp.pad(edge_attr, ((0, EP - E), (0, 3)))

    eW8 = jnp.pad(p['embed_W'], ((0, 2), (0, 0)))
    eb = p['embed_b'].reshape(1, 128)
    ve24 = _rep3(p['vec_embed'][0])

    lp0 = p['l0']
    sc, q, vec = _prelude_call(
        f8, xp, eW8, eb, lp0['qW1'], lp0['qb1'].reshape(1, 128),
        lp0['qW2'], lp0['qb2'].reshape(1, 128), ve24)

    xs, xd = _make_gather2(CE, 128, 128, NP_, NP_)(xp, src2, xp, dst2)
    egeo = _geo_call(xs, xd, ea8)
    sc = sc + 0.0 * xs[0, 0]

    gather_sq = _make_gather2(CE, 128, 128, NP_, NP_)
    scatter_m = _make_scatter(128)
    zm = jnp.zeros((NP_ // NS, 128), F32)

    for l in range(L):
        lp = p['l%d' % l]
        sg, qg = gather_sq(sc, src2, q, dst2)
        W1s = jnp.concatenate([lp['kW1'][:D], lp['vW1'][:D]], axis=1)
        W1e = jnp.zeros((16, 2 * D), F32)
        W1e = W1e.at[3, :D].set(lp['kW1'][D + ED]).at[3, D:].set(lp['vW1'][D + ED])
        W1e = W1e.at[4:9, :D].set(lp['kW1'][D:D + ED]).at[4:9, D:].set(lp['vW1'][D:D + ED])
        b1 = jnp.concatenate([lp['kb1'], lp['vb1']]).reshape(1, 256)
        gWp = jnp.pad(lp['gW'], ((0, 0), (0, 120)))
        gbp = jnp.pad(lp['gb'], (0, 120)).reshape(1, 128)
        mrows, rrows = _edge_call(
            sg, qg, egeo, W1s, W1e, b1,
            lp['kW2'], lp['kb2'].reshape(1, 128),
            lp['vW2'], lp['vb2'].reshape(1, 128), gWp, gbp)
        om = scatter_m(mrows, dst2, zm)
        rrows_dep = rrows + 0.0 * om[0, 0, 0]
        orr = scatter_m(rrows_dep, dst2, zm)[:, :, :32]
        lpn = p['l%d' % min(l + 1, L - 1)]
        sc, q, vec = _node_call(
            om[0], om[1], orr[0], orr[1], sc, vec,
            lp['oW1'], lp['ob1'].reshape(1, 128),
            lp['oW2'], lp['ob2'].reshape(1, 128),
            lpn['qW1'], lpn['qb1'].reshape(1, 128),
            lpn['qW2'], lpn['qb2'].reshape(1, 128),
            _rep3(lp['vscale']))

    feat = _featnode_call(
        sc, vec, p['nmW1'][:D], jnp.pad(p['nmW1'][D:], ((0, 24), (0, 0))),
        p['nmb1'].reshape(1, 128), p['nmW2'], p['nmb2'].reshape(1, 128))

    feat = feat.at[NP_ - 1].set(-1e30)
    ss = jnp.searchsorted(graph_ids, jnp.arange(G + 1, dtype=graph_ids.dtype))
    starts, ends = ss[:G], ss[1:]
    karange = jnp.arange(K_POOL)[None, :]
    pidx = jnp.where(karange < (ends - starts)[:, None],
                     jnp.minimum(starts[:, None] + karange, NP_ - 1),
                     NP_ - 1).astype(jnp.int32)
    prows = _make_gather1(G, 128, NP_)(feat, pidx)
    pooled = _pool_call(prows)

    gm2p = jnp.pad(p['gmW2'], ((0, 0), (0, 127)))
    gb2p = jnp.pad(p['gmb2'], (0, 127)).reshape(1, 128)
    out128 = _gmlp_call(pooled, p['gmW1'], p['gmb1'].reshape(1, 128),
                        gm2p, gb2p)
    return out128[:, :1]


# speedup vs baseline: 14.8361x; 1.0618x over previous
import functools

import jax
import jax.numpy as jnp
import numpy as np
from jax import lax
from jax.experimental import pallas as pl
from jax.experimental.pallas import tpu as pltpu
from jax.experimental.pallas import tpu_sc as plsc

N = 10000
E = 320000
D = 128
H = 4
DH = 32
L = 4
ED = 5
VC = 8
G = 512

NP_ = 10240
EP = 327680
CE = EP // 128
BN = 512
BE = 512
NC, NS = 2, 16
NW = NC * NS
K_POOL = 128

F32 = jnp.float32


def _np_consts():
    c = {}
    Pr16 = np.zeros((16, 32), np.float32)
    PrE = np.zeros((16, 128), np.float32)
    for o in range(8):
        for dd in range(3):
            Pr16[dd, 3 * o + dd] = 1
            PrE[dd, 3 * o + dd] = 1
    c['Pr16'] = Pr16
    c['PrE'] = PrE
    M03 = np.zeros((1, 16), np.float32); M03[0, :3] = 1
    Mc3 = np.zeros((1, 16), np.float32); Mc3[0, 3] = 1
    Ones3 = np.zeros((16, 16), np.float32); Ones3[:3, :] = 1
    S8 = np.zeros((8, 16), np.float32)
    for i in range(5):
        S8[i, 4 + i] = 1
    c['M03'], c['Mc3'], c['Ones3'], c['S8'] = M03, Mc3, Ones3, S8
    Gh = np.zeros((128, 128), np.float32)
    Gb = np.zeros((128, 128), np.float32)
    for h in range(4):
        Gh[h * 32:(h + 1) * 32, h] = 1
        Gb[h, h * 32:(h + 1) * 32] = 1
    c['Gh'], c['Gb'] = Gh, Gb
    Pg = np.zeros((128, 128), np.float32)
    for o in range(8):
        for dd in range(3):
            Pg[o, 3 * o + dd] = 1
    c['Pg'] = Pg
    Db32 = np.zeros((32, 128), np.float32)
    for h in range(4):
        Db32[h, h * 32:(h + 1) * 32] = 1
    c['Db32'] = Db32
    Dv32 = np.zeros((32, 32), np.float32)
    for j in range(24):
        Dv32[4 + j, j] = 1
    c['Dv32'] = Dv32
    Gmat = np.zeros((32, 32), np.float32)
    Gf = np.zeros((32, 32), np.float32)
    for a in range(24):
        Gf[a, a // 3] = 1
        for b in range(24):
            if a // 3 == b // 3:
                Gmat[a, b] = 1
    c['Gmat'], c['Gf'] = Gmat, Gf
    return c


_C = _np_consts()


def _ln(x):
    m = jnp.mean(x, axis=-1, keepdims=True)
    v = jnp.mean((x - m) * (x - m), axis=-1, keepdims=True)
    return (x - m) / jnp.sqrt(v + 1e-5)


def _dot(a, b):
    return jnp.dot(a, b, preferred_element_type=F32)


def _full(spec_shape):
    return pl.BlockSpec(spec_shape, lambda i: tuple(0 for _ in spec_shape))


def _rows(bs, w):
    return pl.BlockSpec((bs, w), lambda i: (i, 0))


def _prelude_body(f8, xp, eW8, eb, qW1, qb1, qW2, qb2, Pr16, ve24,
                  sc_o, q_o, vec_o):
    sc = _dot(f8[...], eW8[...]) + eb[...]
    sc_o[...] = sc
    q_o[...] = _dot(jax.nn.relu(_ln(_dot(sc, qW1[...]) + qb1[...])),
                    qW2[...]) + qb2[...]
    vec_o[...] = _dot(xp[:, :16], Pr16[...]) * ve24[...]


def _prelude_call(f8, xp, eW8, eb, qW1, qb1, qW2, qb2, ve24):
    n = NP_ // BN
    return pl.pallas_call(
        _prelude_body,
        grid=(n,),
        in_specs=[_rows(BN, 8), _rows(BN, 128), _full((8, 128)), _full((1, 128)),
                  _full((128, 128)), _full((1, 128)), _full((128, 128)),
                  _full((1, 128)), _full((16, 32)), _full((1, 32))],
        out_specs=[_rows(BN, 128), _rows(BN, 128), _rows(BN, 32)],
        out_shape=[jax.ShapeDtypeStruct((NP_, 128), F32),
                   jax.ShapeDtypeStruct((NP_, 128), F32),
                   jax.ShapeDtypeStruct((NP_, 32), F32)],
    )(f8, xp, eW8, eb, qW1, qb1, qW2, qb2, jnp.asarray(_C['Pr16']), ve24)


def _geo_body(xs, xd, ea8, M03, Mc3, Ones3, S8, out):
    rel = xs[:, :16] - xd[:, :16]
    d2b = _dot(rel * rel, Ones3[...])
    dist = jnp.sqrt(d2b + 1e-12)
    ru = rel / (dist + 1e-8)
    out[...] = ru * M03[...] + dist * Mc3[...] + _dot(ea8[...], S8[...])


def _geo_call(xs, xd, ea8):
    n = EP // BE
    return pl.pallas_call(
        _geo_body,
        grid=(n,),
        in_specs=[_rows(BE, 128), _rows(BE, 128), _rows(BE, 8),
                  _full((1, 16)), _full((1, 16)), _full((16, 16)),
                  _full((8, 16))],
        out_specs=[_rows(BE, 16)],
        out_shape=[jax.ShapeDtypeStruct((EP, 16), F32)],
    )(xs, xd, ea8, jnp.asarray(_C['M03']), jnp.asarray(_C['Mc3']),
      jnp.asarray(_C['Ones3']), jnp.asarray(_C['S8']))[0]


def _edge_body(sg, qg, egeo, W1s, W1e, b1, kW2, kb2, vW2, vb2, gWp, gbp,
               Gh, Gb, Pg, PrE, m_o, r_o):
    kv = _dot(sg[...], W1s[...]) + _dot(egeo[...], W1e[...]) + b1[...]
    hk = jax.nn.relu(_ln(kv[:, :128]))
    hv = jax.nn.relu(_ln(kv[:, 128:]))
    k = _dot(hk, kW2[...]) + kb2[...]
    v = _dot(hv, vW2[...]) + vb2[...]
    ex128 = jnp.exp(_dot(qg[...] * k, Gh[...]) * (1.0 / np.sqrt(DH)))
    m_o[...] = _dot(ex128, Gb[...]) * v
    gate = jnp.tanh(_dot(v, gWp[...]) + gbp[...])
    vm = _dot(gate, Pg[...]) * _dot(egeo[...], PrE[...])
    r_o[...] = jnp.concatenate(
        [ex128[:, :4], vm[:, :24], jnp.zeros((BE, 100), F32)], axis=1)


def _edge_call(sg, qg, egeo, W1s, W1e, b1, kW2, kb2, vW2, vb2, gWp, gbp):
    n = EP // BE
    return pl.pallas_call(
        _edge_body,
        grid=(n,),
        in_specs=[_rows(BE, 128), _rows(BE, 128), _rows(BE, 16),
                  _full((128, 256)), _full((16, 256)), _full((1, 256)),
                  _full((128, 128)), _full((1, 128)), _full((128, 128)),
                  _full((1, 128)), _full((128, 128)), _full((1, 128)),
                  _full((128, 128)), _full((128, 128)), _full((128, 128)),
                  _full((16, 128))],
        out_specs=[_rows(BE, 128), _rows(BE, 128)],
        out_shape=[jax.ShapeDtypeStruct((EP, 128), F32),
                   jax.ShapeDtypeStruct((EP, 128), F32)],
    )(sg, qg, egeo, W1s, W1e, b1, kW2, kb2, vW2, vb2, gWp, gbp,
      jnp.asarray(_C['Gh']), jnp.asarray(_C['Gb']), jnp.asarray(_C['Pg']),
      jnp.asarray(_C['PrE']))


def _node_body(a0m, a1m, a0r, a1r, sc, vec, oW1, ob1, oW2, ob2,
               qW1, qb1, qW2, qb2, Db32, Dv32, Gmat, vsc24,
               sc_o, q_o, vec_o):
    msum = a0m[...] + a1m[...]
    s32 = a0r[...] + a1r[...]
    den = _dot(s32, Db32[...])
    agg = msum / (den + 1e-9)
    o1 = jax.nn.relu(_ln(_dot(agg, oW1[...]) + ob1[...]))
    sc2 = _ln(sc[...] + _dot(o1, oW2[...]) + ob2[...])
    sc_o[...] = sc2
    q_o[...] = _dot(jax.nn.relu(_ln(_dot(sc2, qW1[...]) + qb1[...])),
                    qW2[...]) + qb2[...]
    vec2 = vec[...] + _dot(s32, Dv32[...])
    gs = _dot(vec2 * vec2, Gmat[...])
    vec_o[...] = vec2 * vsc24[...] / (1.0 + jnp.sqrt(gs + 1e-12))


def _node_call(a0m, a1m, a0r, a1r, sc, vec, oW1, ob1, oW2, ob2,
               qW1, qb1, qW2, qb2, vsc24):
    n = NP_ // BN
    return pl.pallas_call(
        _node_body,
        grid=(n,),
        in_specs=[_rows(BN, 128), _rows(BN, 128), _rows(BN, 32),
                  _rows(BN, 32), _rows(BN, 128), _rows(BN, 32),
                  _full((128, 128)), _full((1, 128)), _full((128, 128)),
                  _full((1, 128)), _full((128, 128)), _full((1, 128)),
                  _full((128, 128)), _full((1, 128)),
                  _full((32, 128)), _full((32, 32)), _full((32, 32)),
                  _full((1, 32))],
        out_specs=[_rows(BN, 128), _rows(BN, 128), _rows(BN, 32)],
        out_shape=[jax.ShapeDtypeStruct((NP_, 128), F32),
                   jax.ShapeDtypeStruct((NP_, 128), F32),
                   jax.ShapeDtypeStruct((NP_, 32), F32)],
    )(a0m, a1m, a0r, a1r, sc, vec, oW1, ob1, oW2, ob2, qW1, qb1, qW2, qb2,
      jnp.asarray(_C['Db32']), jnp.asarray(_C['Dv32']),
      jnp.asarray(_C['Gmat']), vsc24)


def _featnode_body(sc, vec, W1s, W1v, b1, W2, b2, Gf, out):
    gs8 = _dot(vec[...] * vec[...], Gf[...])
    vinv = jnp.sqrt(gs8 + 1e-12)
    h1 = _dot(sc[...], W1s[...]) + _dot(vinv, W1v[...]) + b1[...]
    out[...] = _dot(jax.nn.relu(_ln(h1)), W2[...]) + b2[...] + sc[...]


def _featnode_call(sc, vec, W1s, W1v, b1, W2, b2):
    n = NP_ // BN
    return pl.pallas_call(
        _featnode_body,
        grid=(n,),
        in_specs=[_rows(BN, 128), _rows(BN, 32), _full((128, 128)),
                  _full((32, 128)), _full((1, 128)), _full((128, 128)),
                  _full((1, 128)), _full((32, 32))],
        out_specs=[_rows(BN, 128)],
        out_shape=[jax.ShapeDtypeStruct((NP_, 128), F32)],
    )(sc, vec, W1s, W1v, b1, W2, b2, jnp.asarray(_C['Gf']))[0]


def _pool_body(prows, out):
    for g in range(8):
        seg = prows[g * K_POOL:(g + 1) * K_POOL, :]
        mx = jnp.max(seg, axis=0, keepdims=True)
        out[g:g + 1, :] = jnp.where(mx < -1e29, 0.0, mx)


def _pool_call(prows):
    n = G // 8
    return pl.pallas_call(
        _pool_body,
        grid=(n,),
        in_specs=[pl.BlockSpec((8 * K_POOL, 128), lambda i: (i, 0))],
        out_specs=[_rows(8, 128)],
        out_shape=[jax.ShapeDtypeStruct((G, 128), F32)],
    )(prows)[0]


def _gmlp_body(pooled, W1, b1, W2p, b2p, out):
    h = jax.nn.relu(_ln(_dot(pooled[...], W1[...]) + b1[...]))
    out[...] = _dot(h, W2p[...]) + b2p[...]


def _gmlp_call(pooled, W1, b1, W2p, b2p):
    return pl.pallas_call(
        _gmlp_body,
        grid=(1,),
        in_specs=[_rows(G, 128), _full((128, 128)), _full((1, 128)),
                  _full((128, 128)), _full((1, 128))],
        out_specs=[_rows(G, 128)],
        out_shape=[jax.ShapeDtypeStruct((G, 128), F32)],
    )(pooled, W1, b1, W2p, b2p)[0]


def _mesh():
    return plsc.VectorSubcoreMesh(core_axis_name="c", subcore_axis_name="s",
                                  num_cores=NC, num_subcores=NS)


@functools.cache
def _make_gather2(n_idx_rows, wa, wb, rows_a, rows_b):
    per_w = n_idx_rows // NW

    n2 = per_w // 2

    @functools.partial(
        pl.kernel,
        out_type=(jax.ShapeDtypeStruct((n_idx_rows * 128, wa), F32),
                  jax.ShapeDtypeStruct((n_idx_rows * 128, wb), F32)),
        mesh=_mesh(),
        scratch_types=[pltpu.VMEM((per_w, 128), jnp.int32),
                       pltpu.VMEM((per_w, 128), jnp.int32),
                       pltpu.VMEM((128, wa), F32),
                       pltpu.VMEM((128, wa), F32),
                       pltpu.VMEM((128, wb), F32),
                       pltpu.VMEM((128, wb), F32),
                       pltpu.SemaphoreType.DMA,
                       pltpu.SemaphoreType.DMA,
                       pltpu.SemaphoreType.DMA,
                       pltpu.SemaphoreType.DMA],
    )
    def gather2(ta, ia, tb, ib, oa, ob, iva, ivb, ra0, ra1, rb0, rb1,
                sa0, sa1, sb0, sb1):
        wid = lax.axis_index("c") * NS + lax.axis_index("s")
        base = wid * per_w
        pltpu.sync_copy(ia.at[pl.ds(base, per_w)], iva)
        pltpu.sync_copy(ib.at[pl.ds(base, per_w)], ivb)
        pltpu.async_copy(ta.at[iva.at[0]], ra0, sa0)
        pltpu.async_copy(tb.at[ivb.at[0]], rb0, sb0)

        def body(j2, carry):
            j = 2 * j2
            pltpu.make_async_copy(ta.at[iva.at[j]], ra0, sa0).wait()
            pltpu.make_async_copy(tb.at[ivb.at[j]], rb0, sb0).wait()
            pltpu.async_copy(ta.at[iva.at[j + 1]], ra1, sa1)
            pltpu.async_copy(tb.at[ivb.at[j + 1]], rb1, sb1)
            pltpu.sync_copy(ra0, oa.at[pl.ds((base + j) * 128, 128)])
            pltpu.sync_copy(rb0, ob.at[pl.ds((base + j) * 128, 128)])
            pltpu.make_async_copy(ta.at[iva.at[j + 1]], ra1, sa1).wait()
            pltpu.make_async_copy(tb.at[ivb.at[j + 1]], rb1, sb1).wait()

            @pl.when(j2 < n2 - 1)
            def _():
                pltpu.async_copy(ta.at[iva.at[j + 2]], ra0, sa0)
                pltpu.async_copy(tb.at[ivb.at[j + 2]], rb0, sb0)

            pltpu.sync_copy(ra1, oa.at[pl.ds((base + j + 1) * 128, 128)])
            pltpu.sync_copy(rb1, ob.at[pl.ds((base + j + 1) * 128, 128)])
            return carry

        lax.fori_loop(0, n2, body, 0)

    return gather2


@functools.cache
def _make_gather1(n_idx_rows, w, rows_t):
    per_w = n_idx_rows // NW

    @functools.partial(
        pl.kernel,
        out_type=jax.ShapeDtypeStruct((n_idx_rows * 128, w), F32),
        mesh=_mesh(),
        scratch_types=[pltpu.VMEM((per_w, 128), jnp.int32),
                       pltpu.VMEM((128, w), F32),
                       pltpu.SemaphoreType.DMA],
    )
    def gather1(tab, idx, out, iv, rv, sem):
        wid = lax.axis_index("c") * NS + lax.axis_index("s")
        base = wid * per_w
        pltpu.sync_copy(idx.at[pl.ds(base, per_w)], iv)

        def body(j, carry):
            pltpu.async_copy(tab.at[iv.at[j]], rv, sem).wait()
            pltpu.sync_copy(rv, out.at[pl.ds((base + j) * 128, 128)])
            return carry

        lax.fori_loop(0, per_w, body, 0)

    return gather1


@functools.cache
def _make_scatter(w):
    per_w = CE // NW
    stripe = NP_ // NS

    @functools.partial(
        pl.kernel,
        out_type=jax.ShapeDtypeStruct((NC, NP_, w), F32),
        mesh=_mesh(),
        scratch_types=[pltpu.VMEM((per_w, 128), jnp.int32),
                       pltpu.VMEM((128, w), F32),
                       pltpu.VMEM((128, w), F32),
                       pltpu.VMEM_SHARED((NP_, w), F32),
                       pltpu.SemaphoreType.DMA,
                       pltpu.SemaphoreType.DMA],
    )
    def scatter(mrows, didx, zm, om, iv, mv0, mv1, accm, s0, s1):
        cid = lax.axis_index("c")
        sid = lax.axis_index("s")
        wid = cid * NS + sid
        base = wid * per_w
        pltpu.sync_copy(zm, accm.at[pl.ds(sid * stripe, stripe)])
        plsc.subcore_barrier()
        pltpu.sync_copy(didx.at[pl.ds(base, per_w)], iv)
        pltpu.sync_copy(mrows.at[pl.ds(base * 128, 128)], mv0)
        n2 = per_w // 2

        def body(j2, carry):
            j = 2 * j2
            pltpu.async_copy(mrows.at[pl.ds((base + j + 1) * 128, 128)],
                             mv1, s1)
            pltpu.sync_copy(mv0, accm.at[iv.at[j]], add=True)
            pltpu.make_async_copy(
                mrows.at[pl.ds((base + j + 1) * 128, 128)], mv1, s1).wait()

            @pl.when(j2 < n2 - 1)
            def _():
                pltpu.async_copy(
                    mrows.at[pl.ds((base + j + 2) * 128, 128)], mv0, s0)

            pltpu.sync_copy(mv1, accm.at[iv.at[j + 1]], add=True)

            @pl.when(j2 < n2 - 1)
            def _():
                pltpu.make_async_copy(
                    mrows.at[pl.ds((base + j + 2) * 128, 128)], mv0,
                    s0).wait()

            return carry

        lax.fori_loop(0, n2, body, 0)
        plsc.subcore_barrier()
        pltpu.sync_copy(accm.at[pl.ds(sid * stripe, stripe)],
                        om.at[cid, pl.ds(sid * stripe, stripe)])

    return scatter


def _rep3(v8):
    return jnp.pad(jnp.repeat(v8, 3), (0, 8)).reshape(1, 32)


def kernel(node_f, node_x, edge_index, edge_attr, graph_ids, params):
    p = params
    f6 = node_f[..., 0]
    f6 = f6.at[:, 5].set(f6[:, 5] / 9.0)
    f8 = jnp.pad(f6, ((0, NP_ - N), (0, 2)))
    xp = jnp.pad(node_x, ((0, NP_ - N), (0, 125)))
    src2 = jnp.pad(edge_index[0], (0, EP - E)).reshape(CE, 128).astype(jnp.int32)
    dst_pad = jnp.pad(edge_index[1], (0, EP - E), constant_values=N)
    dst2 = dst_pad.reshape(CE, 128).astype(jnp.int32)
    ea8 = jnp.pad(edge_attr, ((0, EP - E), (0, 3)))

    eW8 = jnp.pad(p['embed_W'], ((0, 2), (0, 0)))
    eb = p['embed_b'].reshape(1, 128)
    ve24 = _rep3(p['vec_embed'][0])

    lp0 = p['l0']
    sc, q, vec = _prelude_call(
        f8, xp, eW8, eb, lp0['qW1'], lp0['qb1'].reshape(1, 128),
        lp0['qW2'], lp0['qb2'].reshape(1, 128), ve24)

    xs, xd = _make_gather2(CE, 128, 128, NP_, NP_)(xp, src2, xp, dst2)
    egeo = _geo_call(xs, xd, ea8)
    sc = sc + 0.0 * xs[0, 0]

    gather_sq = _make_gather2(CE, 128, 128, NP_, NP_)
    scatter_m = _make_scatter(128)
    zm = jnp.zeros((NP_ // NS, 128), F32)

    for l in range(L):
        lp = p['l%d' % l]
        sg, qg = gather_sq(sc, src2, q, dst2)
        W1s = jnp.concatenate([lp['kW1'][:D], lp['vW1'][:D]], axis=1)
        W1e = jnp.zeros((16, 2 * D), F32)
        W1e = W1e.at[3, :D].set(lp['kW1'][D + ED]).at[3, D:].set(lp['vW1'][D + ED])
        W1e = W1e.at[4:9, :D].set(lp['kW1'][D:D + ED]).at[4:9, D:].set(lp['vW1'][D:D + ED])
        b1 = jnp.concatenate([lp['kb1'], lp['vb1']]).reshape(1, 256)
        gWp = jnp.pad(lp['gW'], ((0, 0), (0, 120)))
        gbp = jnp.pad(lp['gb'], (0, 120)).reshape(1, 128)
        mrows, rrows = _edge_call(
            sg, qg, egeo, W1s, W1e, b1,
            lp['kW2'], lp['kb2'].reshape(1, 128),
            lp['vW2'], lp['vb2'].reshape(1, 128), gWp, gbp)
        om = scatter_m(mrows, dst2, zm)
        rrows_dep = rrows + 0.0 * om[0, 0, 0]
        orr = scatter_m(rrows_dep, dst2, zm)[:, :, :32]
        lpn = p['l%d' % min(l + 1, L - 1)]
        sc, q, vec = _node_call(
            om[0], om[1], orr[0], orr[1], sc, vec,
            lp['oW1'], lp['ob1'].reshape(1, 128),
            lp['oW2'], lp['ob2'].reshape(1, 128),
            lpn['qW1'], lpn['qb1'].reshape(1, 128),
            lpn['qW2'], lpn['qb2'].reshape(1, 128),
            _rep3(lp['vscale']))

    feat = _featnode_call(
        sc, vec, p['nmW1'][:D], jnp.pad(p['nmW1'][D:], ((0, 24), (0, 0))),
        p['nmb1'].reshape(1, 128), p['nmW2'], p['nmb2'].reshape(1, 128))

    feat = feat.at[NP_ - 1].set(-1e30)
    ss = jnp.searchsorted(graph_ids, jnp.arange(G + 1, dtype=graph_ids.dtype))
    starts, ends = ss[:G], ss[1:]
    karange = jnp.arange(K_POOL)[None, :]
    pidx = jnp.where(karange < (ends - starts)[:, None],
                     jnp.minimum(starts[:, None] + karange, NP_ - 1),
                     NP_ - 1).astype(jnp.int32)
    prows = _make_gather1(G, 128, NP_)(feat, pidx)
    pooled = _pool_call(prows)

    gm2p = jnp.pad(p['gmW2'], ((0, 0), (0, 127)))
    gb2p = jnp.pad(p['gmb2'], (0, 127)).reshape(1, 128)
    out128 = _gmlp_call(pooled, p['gmW1'], p['gmb1'].reshape(1, 128),
                        gm2p, gb2p)
    return out128[:, :1]
